# baseline (speedup 1.0000x reference)
import functools

import jax
import jax.numpy as jnp
from jax.experimental import pallas as pl
from jax.experimental.pallas import tpu as pltpu

T = 32768
D_MODEL = 768
N_EXPERTS = 8
TOP_K = 2

BLOCK_T = 2048
N_SPLIT = 3
D_CHUNK = D_MODEL // N_SPLIT


def _router_kernel(h0_ref, h1_ref, h2_ref, wt_ref, mask_ref, weight_ref, logits_ref):
    wt = wt_ref[...]
    dn = (((1,), (0,)), ((), ()))
    logits = jax.lax.dot_general(
        h0_ref[...], wt[:D_CHUNK], dn, preferred_element_type=jnp.float32
    )
    logits += jax.lax.dot_general(
        h1_ref[...], wt[D_CHUNK : 2 * D_CHUNK], dn, preferred_element_type=jnp.float32
    )
    logits += jax.lax.dot_general(
        h2_ref[...], wt[2 * D_CHUNK :], dn, preferred_element_type=jnp.float32
    )
    logits_ref[...] = logits

    lt = logits.T

    m1 = jnp.max(lt, axis=0, keepdims=True)
    e = jnp.exp(lt - m1)
    weight = e / jnp.sum(e, axis=0, keepdims=True)

    eidx = jax.lax.broadcasted_iota(jnp.int32, lt.shape, 0)
    big = jnp.int32(N_EXPERTS)
    i1 = jnp.min(jnp.where(lt == m1, eidx, big), axis=0, keepdims=True)
    neg = jnp.float32(-jnp.inf)
    rest = jnp.where(eidx == i1, neg, lt)
    m2 = jnp.max(rest, axis=0, keepdims=True)
    i2 = jnp.min(jnp.where(rest == m2, eidx, big), axis=0, keepdims=True)
    mask = (eidx == i1) | (eidx == i2)

    mask_ref[...] = mask.astype(jnp.float32).T
    weight_ref[...] = weight.T


@jax.jit
def kernel(h_td, W):
    wt = W.T
    grid = (T // BLOCK_T,)
    out_shape = (
        jax.ShapeDtypeStruct((T, N_EXPERTS), jnp.float32),
        jax.ShapeDtypeStruct((T, N_EXPERTS), jnp.float32),
        jax.ShapeDtypeStruct((T, N_EXPERTS), jnp.float32),
    )
    mask_f, weight, logits = pl.pallas_call(
        _router_kernel,
        grid=grid,
        in_specs=[
            pl.BlockSpec((BLOCK_T, D_CHUNK), lambda i: (i, 0)),
            pl.BlockSpec((BLOCK_T, D_CHUNK), lambda i: (i, 1)),
            pl.BlockSpec((BLOCK_T, D_CHUNK), lambda i: (i, 2)),
            pl.BlockSpec((D_MODEL, N_EXPERTS), lambda i: (0, 0)),
        ],
        out_specs=(
            pl.BlockSpec((BLOCK_T, N_EXPERTS), lambda i: (i, 0)),
            pl.BlockSpec((BLOCK_T, N_EXPERTS), lambda i: (i, 0)),
            pl.BlockSpec((BLOCK_T, N_EXPERTS), lambda i: (i, 0)),
        ),
        out_shape=out_shape,
    )(h_td, h_td, h_td, wt)
    return (mask_f.astype(bool), weight, logits)


# speedup vs baseline: 1.0085x; 1.0085x over previous
import functools

import jax
import jax.numpy as jnp
from jax.experimental import pallas as pl
from jax.experimental.pallas import tpu as pltpu

T = 32768
D_MODEL = 768
N_EXPERTS = 8
TOP_K = 2

CHUNK_T = 512
N_CHUNK = T // CHUNK_T
N_BUF = 12


def _chunk_copy(h_hbm, buf, sems, chunk, slot):
    return pltpu.make_async_copy(
        h_hbm.at[pl.ds(chunk * CHUNK_T, CHUNK_T), :],
        buf.at[slot],
        sems.at[slot],
    )


def _router_kernel(h_hbm, wt_ref, mask_ref, weight_ref, logits_ref, buf, sems):
    i = pl.program_id(0)
    wt = wt_ref[...]

    @pl.when(i == 0)
    def _prologue():
        for b in range(N_BUF):
            _chunk_copy(h_hbm, buf, sems, b, b).start()

    slot = jax.lax.rem(i, N_BUF)
    _chunk_copy(h_hbm, buf, sems, i, slot).wait()
    x = buf[slot]

    logits = jax.lax.dot_general(
        x, wt, (((1,), (0,)), ((), ())), preferred_element_type=jnp.float32
    )
    logits_ref[...] = logits

    lt = logits.T
    m1 = jnp.max(lt, axis=0, keepdims=True)
    e = jnp.exp(lt - m1)
    weight = e / jnp.sum(e, axis=0, keepdims=True)

    eidx = jax.lax.broadcasted_iota(jnp.int32, lt.shape, 0)
    big = jnp.int32(N_EXPERTS)
    i1 = jnp.min(jnp.where(lt == m1, eidx, big), axis=0, keepdims=True)
    neg = jnp.float32(-jnp.inf)
    rest = jnp.where(eidx == i1, neg, lt)
    m2 = jnp.max(rest, axis=0, keepdims=True)
    i2 = jnp.min(jnp.where(rest == m2, eidx, big), axis=0, keepdims=True)
    mask = (eidx == i1) | (eidx == i2)

    mask_ref[...] = mask.astype(jnp.float32).T
    weight_ref[...] = weight.T

    @pl.when(i + N_BUF < N_CHUNK)
    def _prefetch():
        _chunk_copy(h_hbm, buf, sems, i + N_BUF, slot).start()


@jax.jit
def kernel(h_td, W):
    wt = W.T
    out_shape = (
        jax.ShapeDtypeStruct((T, N_EXPERTS), jnp.float32),
        jax.ShapeDtypeStruct((T, N_EXPERTS), jnp.float32),
        jax.ShapeDtypeStruct((T, N_EXPERTS), jnp.float32),
    )
    out_spec = pl.BlockSpec((CHUNK_T, N_EXPERTS), lambda i: (i, 0))
    mask_f, weight, logits = pl.pallas_call(
        _router_kernel,
        grid=(N_CHUNK,),
        in_specs=[
            pl.BlockSpec(memory_space=pl.ANY),
            pl.BlockSpec((D_MODEL, N_EXPERTS), lambda i: (0, 0)),
        ],
        out_specs=(out_spec, out_spec, out_spec),
        out_shape=out_shape,
        scratch_shapes=[
            pltpu.VMEM((N_BUF, CHUNK_T, D_MODEL), jnp.float32),
            pltpu.SemaphoreType.DMA((N_BUF,)),
        ],
    )(h_td, wt)
    return (mask_f.astype(bool), weight, logits)


# speedup vs baseline: 2.2700x; 2.2509x over previous
import functools

import jax
import jax.numpy as jnp
from jax.experimental import pallas as pl
from jax.experimental.pallas import tpu as pltpu

T = 32768
D_MODEL = 768
N_EXPERTS = 8
TOP_K = 2

BLOCK_T = 2048


def _router_kernel(h_ref, wt_ref, mask_ref, weight_ref, logits_ref):
    x = h_ref[...]
    wt = wt_ref[...]
    logits = jax.lax.dot_general(
        x, wt, (((1,), (0,)), ((), ())), preferred_element_type=jnp.float32
    )

    lt = logits.T
    logits_ref[...] = lt

    m1 = jnp.max(lt, axis=0, keepdims=True)
    e = jnp.exp(lt - m1)
    weight = e / jnp.sum(e, axis=0, keepdims=True)

    eidx = jax.lax.broadcasted_iota(jnp.int32, lt.shape, 0)
    big = jnp.int32(N_EXPERTS)
    i1 = jnp.min(jnp.where(lt == m1, eidx, big), axis=0, keepdims=True)
    neg = jnp.float32(-jnp.inf)
    rest = jnp.where(eidx == i1, neg, lt)
    m2 = jnp.max(rest, axis=0, keepdims=True)
    i2 = jnp.min(jnp.where(rest == m2, eidx, big), axis=0, keepdims=True)
    mask = (eidx == i1) | (eidx == i2)

    mask_ref[...] = mask.astype(jnp.float32)
    weight_ref[...] = weight


@jax.jit
def kernel(h_td, W):
    wt = W.T
    grid = (T // BLOCK_T,)
    out_shape = (
        jax.ShapeDtypeStruct((N_EXPERTS, T), jnp.float32),
        jax.ShapeDtypeStruct((N_EXPERTS, T), jnp.float32),
        jax.ShapeDtypeStruct((N_EXPERTS, T), jnp.float32),
    )
    out_spec = pl.BlockSpec((N_EXPERTS, BLOCK_T), lambda i: (0, i))
    mask_f, weight, logits = pl.pallas_call(
        _router_kernel,
        grid=grid,
        in_specs=[
            pl.BlockSpec((BLOCK_T, D_MODEL), lambda i: (i, 0)),
            pl.BlockSpec((D_MODEL, N_EXPERTS), lambda i: (0, 0)),
        ],
        out_specs=(out_spec, out_spec, out_spec),
        out_shape=out_shape,
    )(h_td, wt)
    return (mask_f.T.astype(bool), weight.T, logits.T)
